# Initial kernel scaffold; baseline (speedup 1.0000x reference)
import jax
import jax.numpy as jnp
from jax.experimental import pallas as pl


def kernel(x, W, b, time_table, joint_table, nan_table):
    raise NotImplementedError("write your pallas kernel here")


# speedup vs baseline: 1.9454x; 1.9454x over previous
import jax
import jax.numpy as jnp
from jax.experimental import pallas as pl
from jax.experimental.pallas import tpu as pltpu

_N_T = 200
_N_J = 25
_D_IN = 3
_D_M = 128
_T_BLK = 40


def _body(x_ref, tt_ref, jt_ref, w_ref, b_ref, nan_ref, out_ref):
    xb = x_ref[0]
    nanmask = jnp.isnan(xb)
    m = jnp.any(nanmask, axis=2, keepdims=True)
    xc = jnp.where(nanmask, 0.0, xb)
    tt = tt_ref[...]
    jt = jt_ref[...]
    w = w_ref[...]
    nan0 = nan_ref[0]
    dnan = nan_ref[1] - nan0
    acc = tt + jt + (b_ref[...] + nan0)[None]
    acc = acc + xc[:, :, 0:1] * w[0][None]
    acc = acc + xc[:, :, 1:2] * w[1][None]
    acc = acc + xc[:, :, 2:3] * w[2][None]
    acc = acc + jnp.where(m, dnan[None], 0.0)
    out_ref[0] = acc


def kernel(x, W, b, time_table, joint_table, nan_table):
    B, S, _ = x.shape
    x4 = x.reshape(B, _N_T, _N_J, _D_IN)
    tt3 = time_table.reshape(_N_T, 1, _D_M)
    jt3 = joint_table.reshape(1, _N_J, _D_M)
    w3 = W.T.reshape(_D_IN, 1, _D_M)
    b2 = b.reshape(1, _D_M)
    nan3 = nan_table.reshape(2, 1, _D_M)
    n_tb = _N_T // _T_BLK
    out4 = pl.pallas_call(
        _body,
        grid=(n_tb, B),
        in_specs=[
            pl.BlockSpec((1, _T_BLK, _N_J, _D_IN), lambda i, bb: (bb, i, 0, 0)),
            pl.BlockSpec((_T_BLK, 1, _D_M), lambda i, bb: (i, 0, 0)),
            pl.BlockSpec((1, _N_J, _D_M), lambda i, bb: (0, 0, 0)),
            pl.BlockSpec((_D_IN, 1, _D_M), lambda i, bb: (0, 0, 0)),
            pl.BlockSpec((1, _D_M), lambda i, bb: (0, 0)),
            pl.BlockSpec((2, 1, _D_M), lambda i, bb: (0, 0, 0)),
        ],
        out_specs=pl.BlockSpec((1, _T_BLK, _N_J, _D_M), lambda i, bb: (bb, i, 0, 0)),
        out_shape=jax.ShapeDtypeStruct((B, _N_T, _N_J, _D_M), jnp.float32),
    )(x4, tt3, jt3, w3, b2, nan3)
    return out4.reshape(B, S, _D_M)


# speedup vs baseline: 2.3291x; 1.1972x over previous
import jax
import jax.numpy as jnp
from jax.experimental import pallas as pl
from jax.experimental.pallas import tpu as pltpu

_N_T = 200
_N_J = 25
_D_IN = 3
_D_M = 128
_T_BLK = 200


def _body(x_ref, tt_ref, jt_ref, w_ref, b_ref, nan_ref, out_ref):
    xb = x_ref[0]
    nanmask = jnp.isnan(xb)
    m = jnp.any(nanmask, axis=2, keepdims=True)
    xc = jnp.where(nanmask, 0.0, xb)
    tt = tt_ref[...]
    jt = jt_ref[...]
    w = w_ref[...]
    nan0 = nan_ref[0]
    dnan = nan_ref[1] - nan0
    acc = tt + jt + (b_ref[...] + nan0)[None]
    acc = acc + xc[:, :, 0:1] * w[0][None]
    acc = acc + xc[:, :, 1:2] * w[1][None]
    acc = acc + xc[:, :, 2:3] * w[2][None]
    acc = acc + jnp.where(m, dnan[None], 0.0)
    out_ref[0] = acc


def kernel(x, W, b, time_table, joint_table, nan_table):
    B, S, _ = x.shape
    x4 = x.reshape(B, _N_T, _N_J, _D_IN)
    tt3 = time_table.reshape(_N_T, 1, _D_M)
    jt3 = joint_table.reshape(1, _N_J, _D_M)
    w3 = W.T.reshape(_D_IN, 1, _D_M)
    b2 = b.reshape(1, _D_M)
    nan3 = nan_table.reshape(2, 1, _D_M)
    n_tb = _N_T // _T_BLK
    out4 = pl.pallas_call(
        _body,
        grid=(n_tb, B),
        in_specs=[
            pl.BlockSpec((1, _T_BLK, _N_J, _D_IN), lambda i, bb: (bb, i, 0, 0)),
            pl.BlockSpec((_T_BLK, 1, _D_M), lambda i, bb: (i, 0, 0)),
            pl.BlockSpec((1, _N_J, _D_M), lambda i, bb: (0, 0, 0)),
            pl.BlockSpec((_D_IN, 1, _D_M), lambda i, bb: (0, 0, 0)),
            pl.BlockSpec((1, _D_M), lambda i, bb: (0, 0)),
            pl.BlockSpec((2, 1, _D_M), lambda i, bb: (0, 0, 0)),
        ],
        out_specs=pl.BlockSpec((1, _T_BLK, _N_J, _D_M), lambda i, bb: (bb, i, 0, 0)),
        out_shape=jax.ShapeDtypeStruct((B, _N_T, _N_J, _D_M), jnp.float32),
    )(x4, tt3, jt3, w3, b2, nan3)
    return out4.reshape(B, S, _D_M)
